# Initial kernel scaffold; baseline (speedup 1.0000x reference)
import jax
import jax.numpy as jnp
from jax.experimental import pallas as pl


def kernel(x, A_neg, A_pos, W1, b1, W2, b2, W3, b3, Wg1, bg1, Wg2, bg2, Wg3, bg3, Wg4, bg4, Wg5, bg5, Wg6, bg6):
    raise NotImplementedError("write your pallas kernel here")


# speedup vs baseline: 2928.7787x; 2928.7787x over previous
import functools

import jax
import jax.numpy as jnp
from jax.experimental import pallas as pl

N = 2048
BLK = 256


def _propagate_body(t_ref, ab_ref, out_ref):
    i = pl.program_id(0)
    t = t_ref[...]
    c = jnp.dot(t.astype(jnp.bfloat16), ab_ref[...],
                preferred_element_type=jnp.float32)
    rows = i * BLK + jax.lax.broadcasted_iota(jnp.int32, (BLK, N), 0)
    cols = jax.lax.broadcasted_iota(jnp.int32, (BLK, N), 1)
    new = jnp.where((c > 0.0) & (rows != cols), 1.0, 0.0)
    out_ref[...] = jnp.maximum(t, new)


def _propagate(t, ab_bf):
    return pl.pallas_call(
        _propagate_body,
        grid=(N // BLK,),
        in_specs=[
            pl.BlockSpec((BLK, N), lambda i: (i, 0)),
            pl.BlockSpec((N, N), lambda i: (0, 0)),
        ],
        out_specs=pl.BlockSpec((BLK, N), lambda i: (i, 0)),
        out_shape=jax.ShapeDtypeStruct((N, N), jnp.float32),
    )(t, ab_bf)


def _layer_body(has_linear, relu, weight,
                xi_ref, t_ref, wt_ref, b_ref, wgt_ref, bg_ref, out_ref):
    xi = xi_ref[...]
    if has_linear:
        xlin = jnp.dot(xi, wt_ref[...], preferred_element_type=jnp.float32)
        xlin = xlin + b_ref[...]
    else:
        xlin = xi
    t = t_ref[...]
    deg = jnp.sum(t, axis=1, keepdims=True) + 1.0
    dinv = jax.lax.rsqrt(deg)
    y = jnp.dot(xlin, wgt_ref[...], preferred_element_type=jnp.float32) * dinv
    agg = jnp.dot(t, y, preferred_element_type=jnp.float32) + y
    g = agg * dinv + bg_ref[...]
    if relu:
        g = jnp.maximum(g, 0.0)
    out_ref[...] = xlin + weight * g


def _layer(xi, t, wt, b, wgt, bg, has_linear, relu, weight):
    body = functools.partial(_layer_body, has_linear, relu, weight)
    return pl.pallas_call(
        body,
        out_shape=jax.ShapeDtypeStruct((N, wgt.shape[1]), jnp.float32),
    )(xi, t, wt, b, wgt, bg)


def kernel(x, A_neg, A_pos, W1, b1, W2, b2, W3, b3,
           Wg1, bg1, Wg2, bg2, Wg3, bg3, Wg4, bg4, Wg5, bg5, Wg6, bg6):
    t = A_neg.T
    ab_bf = A_pos.T.astype(jnp.bfloat16)

    w1t = W1.T
    b1r = b1[None, :]
    w2t = jnp.pad(W2.T, ((0, 0), (0, 2)))
    b2r = jnp.pad(b2, (0, 2))[None, :]
    w3t = jnp.pad(W3.T, ((0, 2), (0, 0)))
    b3r = b3[None, :]
    wg1t = Wg1.T
    bg1r = bg1[None, :]
    wg2t = jnp.pad(Wg2.T, ((0, 2), (0, 2)))
    bg2r = jnp.pad(bg2, (0, 2))[None, :]
    wg3t, bg3r = Wg3.T, bg3[None, :]
    wg4t, bg4r = Wg4.T, bg4[None, :]
    wg5t, bg5r = Wg5.T, bg5[None, :]
    wg6t, bg6r = Wg6.T, bg6[None, :]

    x1 = _layer(x, t, w1t, b1r, wg1t, bg1r, True, True, 1.0)
    t = _propagate(t, ab_bf)
    x2 = _layer(x1, t, w2t, b2r, wg2t, bg2r, True, True, 1.0)
    t = _propagate(t, ab_bf)
    x3 = _layer(x2, t, w3t, b3r, wg3t, bg3r, True, True, 0.5)
    t = _propagate(t, ab_bf)
    x4 = _layer(x3, t, x3, b3r, wg4t, bg4r, False, True, 0.5)
    t = _propagate(t, ab_bf)
    x5 = _layer(x4, t, x4, b3r, wg5t, bg5r, False, True, 0.25)
    t = _propagate(t, ab_bf)
    x6 = _layer(x5, t, x5, b3r, wg6t, bg6r, False, False, 0.25)
    return x6
